# baseline (speedup 1.0000x reference)
import functools

import jax
import jax.numpy as jnp
from jax import lax
from jax.experimental import pallas as pl
from jax.experimental.pallas import tpu as pltpu
from jax.experimental.pallas import tpu_sc as plsc

_VOCAB = 1000000
_FEATURES = 32
_BATCH = 16384

_info = plsc.get_sparse_core_info()
_NC, _NS = _info.num_cores, _info.num_subcores
_NW = _NC * _NS
_LANES = 128
_TC_TOTAL = -(-_VOCAB // _LANES)
_WIN_TC = 3
_WIN = _WIN_TC * _LANES
_MAX_TC = -(-_TC_TOTAL // _NW)
_N_ROUNDS = -(-_MAX_TC // _WIN_TC)
_TAIL_LANE = (_VOCAB // _LANES) * _LANES
_TAIL_W = _VOCAB - _TAIL_LANE
_MAX_START = ((_VOCAB - _WIN) // _LANES) * _LANES
_XCH = 2048
_N_XCH = _BATCH // _XCH

_mesh = plsc.VectorSubcoreMesh(core_axis_name="c", subcore_axis_name="s")


@functools.partial(
    pl.kernel,
    mesh=_mesh,
    compiler_params=pltpu.CompilerParams(needs_layout_passes=False),
    out_type=(
        jax.ShapeDtypeStruct((_BATCH, _FEATURES), jnp.float32),
        jax.ShapeDtypeStruct((_BATCH, _FEATURES), jnp.float32),
    ),
    scratch_types=[
        pltpu.VMEM((_XCH,), jnp.int32),
        pltpu.VMEM((_BATCH + 16,), jnp.int32),
        pltpu.VMEM((_BATCH + 16,), jnp.int32),
        pltpu.VMEM((_BATCH + 16,), jnp.int32),
        pltpu.VMEM((_BATCH + 16,), jnp.int32),
        pltpu.VMEM((2, 2, _FEATURES, _WIN), jnp.float32),
        pltpu.VMEM((2, _FEATURES, _TAIL_W), jnp.float32),
        pltpu.VMEM((2, 16, _FEATURES), jnp.float32),
        pltpu.SemaphoreType.DMA,
        pltpu.SemaphoreType.DMA,
    ],
)
def _dual_gather(real_hbm, imag_hbm, x_hbm, tail_r_hbm, tail_i_hbm,
                 out_r_hbm, out_i_hbm,
                 xch, sval, spos, rbv, rbp, win, tailbuf, slab,
                 sem_w, sem_o):
    wid = lax.axis_index("s") * _NC + lax.axis_index("c")
    lo_tc = (_TC_TOTAL * wid) // _NW
    hi_tc = (_TC_TOTAL * (wid + 1)) // _NW
    lo_lane = lo_tc * _LANES
    hi_lane = jnp.minimum(hi_tc * _LANES, _VOCAB)

    iota = lax.iota(jnp.int32, 16)
    lo16 = jnp.full((16,), lo_lane, jnp.int32)
    hi16 = jnp.full((16,), hi_lane, jnp.int32)

    def collect_chunk(ch, off):
        pltpu.sync_copy(x_hbm.at[pl.ds(ch * _XCH, _XCH)], xch)

        def collect(g, off):
            s = xch[pl.ds(g * 16, 16)]
            m = (s >= lo16) & (s < hi16)
            mi = jnp.where(m, 1, 0)
            c = lax.cumsum(mi)
            pos = jnp.where(m, off + c - mi, _BATCH + iota)
            plsc.store_scatter(sval, [pos], s)
            plsc.store_scatter(spos, [pos], iota + (ch * _XCH + g * 16))
            return off + c[15]

        return lax.fori_loop(0, _XCH // 16, collect, off)

    n_hits = lax.fori_loop(0, _N_XCH, collect_chunk, 0)
    n_hv = (n_hits + 15) >> 4

    def drain_rows(n, out_hbm):
        for u in range(16):
            @pl.when(u < n)
            def _():
                pltpu.make_async_copy(slab.at[0, 0], out_hbm.at[0],
                                      sem_o).wait()

    def fire_windows(r, slot):
        start = pl.multiple_of(
            jnp.minimum(lo_lane + r * _WIN, _MAX_START), _LANES)
        pltpu.async_copy(real_hbm.at[:, pl.ds(start, _WIN)],
                         win.at[0, slot], sem_w)
        pltpu.async_copy(imag_hbm.at[:, pl.ds(start, _WIN)],
                         win.at[1, slot], sem_w)

    def wait_windows(slot):
        for t in range(2):
            pltpu.make_async_copy(real_hbm.at[:, pl.ds(0, _WIN)],
                                  win.at[t, slot], sem_w).wait()

    def process_round(gather_from, start, vlo, vhi, state):
        start16 = jnp.full((16,), start, jnp.int32)
        vlo16 = jnp.full((16,), vlo, jnp.int32)
        vhi16 = jnp.full((16,), vhi, jnp.int32)

        def compress(g, off):
            valid = (g * 16 + iota) < n_hits
            s = sval[pl.ds(g * 16, 16)]
            m = valid & (s >= vlo16) & (s < vhi16)
            mi = jnp.where(m, 1, 0)
            c = lax.cumsum(mi)
            pos = jnp.where(m, off + c - mi, _BATCH + iota)
            plsc.store_scatter(rbv, [pos], s - start16)
            p = spos[pl.ds(g * 16, 16)]
            plsc.store_scatter(rbp, [pos], p)
            return off + c[15]

        cnt_r = lax.fori_loop(0, n_hv, compress, 0)
        n_grp = (cnt_r + 15) >> 4

        def group(k, st):
            c0, c1 = st
            m = (k * 16 + iota) < cnt_r
            col = jnp.where(m, rbv[pl.ds(k * 16, 16)], 0)
            pvec = rbp[pl.ds(k * 16, 16)]
            grp = jnp.minimum(cnt_r - k * 16, 16)
            for t, out_hbm in ((0, out_r_hbm), (1, out_i_hbm)):
                drain_rows(c0 if t == 0 else c1, out_hbm)
                ring16 = jnp.full((16,), t, jnp.int32)
                for f in range(_FEATURES):
                    f16 = jnp.full((16,), f, jnp.int32)
                    v = gather_from(t, f16, col)
                    plsc.store_scatter(slab, [ring16, iota, f16], v)
                for u in range(16):
                    @pl.when(u < grp)
                    def _():
                        pltpu.async_copy(slab.at[t, u], out_hbm.at[pvec[u]],
                                         sem_o)
            return (grp, grp)

        return lax.fori_loop(0, n_grp, group, state)

    state = (0, 0)
    fire_windows(0, 0)

    def round_body(r, st):
        @pl.when(r + 1 < _N_ROUNDS)
        def _():
            fire_windows(r + 1, (r + 1) & 1)
        slot = r & 1
        wait_windows(slot)
        start = jnp.minimum(lo_lane + r * _WIN, _MAX_START)
        vlo = lo_lane + r * _WIN
        vhi = jnp.minimum(jnp.minimum(vlo + _WIN, hi_lane), _TAIL_LANE)
        slot16 = jnp.full((16,), slot, jnp.int32)

        def gather_from(t, f16, col):
            t16 = jnp.full((16,), t, jnp.int32)
            return plsc.load_gather(win, [t16, slot16, f16, col])

        return process_round(gather_from, start, vlo, vhi, st)

    state = lax.fori_loop(0, _N_ROUNDS, round_body, state)

    pltpu.sync_copy(tail_r_hbm, tailbuf.at[0])
    pltpu.sync_copy(tail_i_hbm, tailbuf.at[1])

    def gather_tail(t, f16, col):
        t16 = jnp.full((16,), t, jnp.int32)
        return plsc.load_gather(tailbuf, [t16, f16, col])

    state = process_round(gather_tail, _TAIL_LANE, _TAIL_LANE, _VOCAB, state)

    c0, c1 = state
    drain_rows(c0, out_r_hbm)
    drain_rows(c1, out_i_hbm)


def kernel(real_table, imag_table, x):
    tr = real_table.T
    ti = imag_table.T
    real_embed, imag_embed = _dual_gather(
        tr, ti, x.astype(jnp.int32),
        tr[:, _TAIL_LANE:], ti[:, _TAIL_LANE:])
    return (real_embed, imag_embed)


# speedup vs baseline: 1.2143x; 1.2143x over previous
import functools

import jax
import jax.numpy as jnp
from jax import lax
from jax.experimental import pallas as pl
from jax.experimental.pallas import tpu as pltpu
from jax.experimental.pallas import tpu_sc as plsc

_VOCAB = 1000000
_FEATURES = 32
_BATCH = 16384

_info = plsc.get_sparse_core_info()
_NC, _NS = _info.num_cores, _info.num_subcores
_NW = _NC * _NS
_LANES = 128
_TC_TOTAL = -(-_VOCAB // _LANES)
_WIN_TC = 5
_WIN = _WIN_TC * _LANES
_MAX_TC = -(-_TC_TOTAL // _NW)
_N_ROUNDS = -(-_MAX_TC // _WIN_TC)
_TAIL_LANE = (_VOCAB // _LANES) * _LANES
_TAIL_W = _VOCAB - _TAIL_LANE
_MAX_START = ((_VOCAB - _WIN) // _LANES) * _LANES
_XCH = 2048
_N_XCH = _BATCH // _XCH

_mesh = plsc.VectorSubcoreMesh(core_axis_name="c", subcore_axis_name="s")


@functools.partial(
    pl.kernel,
    mesh=_mesh,
    compiler_params=pltpu.CompilerParams(needs_layout_passes=False),
    out_type=(
        jax.ShapeDtypeStruct((_BATCH, _FEATURES), jnp.float32),
        jax.ShapeDtypeStruct((_BATCH, _FEATURES), jnp.float32),
    ),
    scratch_types=[
        pltpu.VMEM((_XCH,), jnp.int32),
        pltpu.VMEM((_BATCH + 16,), jnp.int32),
        pltpu.VMEM((_BATCH + 16,), jnp.int32),
        pltpu.VMEM((2, 2, _FEATURES, _WIN), jnp.float32),
        pltpu.VMEM((2, _FEATURES, _TAIL_W), jnp.float32),
        pltpu.VMEM((2, 16, _FEATURES), jnp.float32),
        pltpu.SemaphoreType.DMA,
        pltpu.SemaphoreType.DMA,
    ],
)
def _dual_gather(real_hbm, imag_hbm, x_hbm, tail_r_hbm, tail_i_hbm,
                 out_r_hbm, out_i_hbm,
                 xch, sh, rb, win, tailbuf, slab,
                 sem_w, sem_o):
    wid = lax.axis_index("s") * _NC + lax.axis_index("c")
    lo_tc = (_TC_TOTAL * wid) // _NW
    hi_tc = (_TC_TOTAL * (wid + 1)) // _NW
    lo_lane = lo_tc * _LANES
    hi_lane = jnp.minimum(hi_tc * _LANES, _VOCAB)

    iota = lax.iota(jnp.int32, 16)
    lo16 = jnp.full((16,), lo_lane, jnp.int32)
    hi16 = jnp.full((16,), hi_lane, jnp.int32)

    def collect_chunk(ch, off):
        pltpu.sync_copy(x_hbm.at[pl.ds(ch * _XCH, _XCH)], xch)

        def collect(g, off):
            s = xch[pl.ds(g * 16, 16)]
            m = (s >= lo16) & (s < hi16)
            mi = jnp.where(m, 1, 0)
            c = lax.cumsum(mi)
            pos = jnp.where(m, off + c - mi, _BATCH + iota)
            packed = ((s - lo16) << 14) | (iota + (ch * _XCH + g * 16))
            plsc.store_scatter(sh, [pos], packed)
            return off + c[15]

        return lax.fori_loop(0, _XCH // 16, collect, off)

    n_hits = lax.fori_loop(0, _N_XCH, collect_chunk, 0)
    n_hv = (n_hits + 15) >> 4

    def drain_rows(n, out_hbm):
        for u in range(16):
            @pl.when(u < n)
            def _():
                pltpu.make_async_copy(slab.at[0, 0], out_hbm.at[0],
                                      sem_o).wait()

    def fire_windows(r, slot):
        start = pl.multiple_of(
            jnp.minimum(lo_lane + r * _WIN, _MAX_START), _LANES)
        pltpu.async_copy(real_hbm.at[:, pl.ds(start, _WIN)],
                         win.at[0, slot], sem_w)
        pltpu.async_copy(imag_hbm.at[:, pl.ds(start, _WIN)],
                         win.at[1, slot], sem_w)

    def wait_windows(slot):
        for t in range(2):
            pltpu.make_async_copy(real_hbm.at[:, pl.ds(0, _WIN)],
                                  win.at[t, slot], sem_w).wait()

    def process_round(gather_from, start, vlo, vhi, state):
        plo16 = jnp.full((16,), (vlo - lo_lane) << 14, jnp.int32)
        phi16 = jnp.full((16,), (vhi - lo_lane) << 14, jnp.int32)
        shift16 = jnp.full((16,), (start - lo_lane) << 14, jnp.int32)

        def compress(g, off):
            valid = (g * 16 + iota) < n_hits
            p = sh[pl.ds(g * 16, 16)]
            m = valid & (p >= plo16) & (p < phi16)
            mi = jnp.where(m, 1, 0)
            c = lax.cumsum(mi)
            pos = jnp.where(m, off + c - mi, _BATCH + iota)
            plsc.store_scatter(rb, [pos], p - shift16)
            return off + c[15]

        cnt_r = lax.fori_loop(0, n_hv, compress, 0)
        n_grp = (cnt_r + 15) >> 4

        def group(k, st):
            c0, c1 = st
            m = (k * 16 + iota) < cnt_r
            p = rb[pl.ds(k * 16, 16)]
            col = jnp.where(m, p >> 14, 0)
            pvec = p & 16383
            grp = jnp.minimum(cnt_r - k * 16, 16)
            for t, out_hbm in ((0, out_r_hbm), (1, out_i_hbm)):
                drain_rows(c0 if t == 0 else c1, out_hbm)
                ring16 = jnp.full((16,), t, jnp.int32)
                for f in range(_FEATURES):
                    f16 = jnp.full((16,), f, jnp.int32)
                    v = gather_from(t, f16, col)
                    plsc.store_scatter(slab, [ring16, iota, f16], v)
                for u in range(16):
                    @pl.when(u < grp)
                    def _():
                        pltpu.async_copy(slab.at[t, u], out_hbm.at[pvec[u]],
                                         sem_o)
            return (grp, grp)

        return lax.fori_loop(0, n_grp, group, state)

    state = (0, 0)
    fire_windows(0, 0)

    def round_body(r, st):
        @pl.when(r + 1 < _N_ROUNDS)
        def _():
            fire_windows(r + 1, (r + 1) & 1)
        slot = r & 1
        wait_windows(slot)
        start = jnp.minimum(lo_lane + r * _WIN, _MAX_START)
        vlo = lo_lane + r * _WIN
        vhi = jnp.minimum(jnp.minimum(vlo + _WIN, hi_lane), _TAIL_LANE)
        slot16 = jnp.full((16,), slot, jnp.int32)

        def gather_from(t, f16, col):
            t16 = jnp.full((16,), t, jnp.int32)
            return plsc.load_gather(win, [t16, slot16, f16, col])

        return process_round(gather_from, start, vlo, vhi, st)

    state = lax.fori_loop(0, _N_ROUNDS, round_body, state)

    pltpu.sync_copy(tail_r_hbm, tailbuf.at[0])
    pltpu.sync_copy(tail_i_hbm, tailbuf.at[1])

    def gather_tail(t, f16, col):
        t16 = jnp.full((16,), t, jnp.int32)
        return plsc.load_gather(tailbuf, [t16, f16, col])

    state = process_round(gather_tail, _TAIL_LANE, _TAIL_LANE, _VOCAB, state)

    c0, c1 = state
    drain_rows(c0, out_r_hbm)
    drain_rows(c1, out_i_hbm)


def kernel(real_table, imag_table, x):
    tr = real_table.T
    ti = imag_table.T
    real_embed, imag_embed = _dual_gather(
        tr, ti, x.astype(jnp.int32),
        tr[:, _TAIL_LANE:], ti[:, _TAIL_LANE:])
    return (real_embed, imag_embed)
